# Initial kernel scaffold; baseline (speedup 1.0000x reference)
import jax
import jax.numpy as jnp
from jax.experimental import pallas as pl


def kernel(h, e, edge_index, W_ph, b_ph, W_pe, b_pe, W_n1, b_n1, W_n2, b_n2, W_e1, b_e1, W_e2, b_e2, g_h, bt_h, g_e, bt_e):
    raise NotImplementedError("write your pallas kernel here")


# speedup vs baseline: 4.7553x; 4.7553x over previous
import functools

import jax
import jax.numpy as jnp
from jax import lax
from jax.experimental import pallas as pl
from jax.experimental.pallas import tpu as pltpu
from jax.experimental.pallas import tpu_sc as plsc

EPS = 1e-5
F32 = jnp.float32

NUM_CORES = 2
NUM_SUBCORES = 16
LANES = 16
WIN = 128


def _prelin_body(x_ref, w_ref, b_ref, o_ref):
    o_ref[...] = (
        jnp.dot(x_ref[...], w_ref[...], preferred_element_type=F32) + b_ref[...]
    )


def _prelin(x, w, b, block):
    n, d = x.shape
    assert n % block == 0
    return pl.pallas_call(
        _prelin_body,
        grid=(n // block,),
        in_specs=[
            pl.BlockSpec((block, d), lambda i: (i, 0)),
            pl.BlockSpec((d, d), lambda i: (0, 0)),
            pl.BlockSpec((1, d), lambda i: (0, 0)),
        ],
        out_specs=pl.BlockSpec((block, d), lambda i: (i, 0)),
        out_shape=jax.ShapeDtypeStruct((n, d), F32),
    )(x, w, b)


def _layernorm(x, g, b):
    mu = jnp.mean(x, axis=-1, keepdims=True)
    xc = x - mu
    var = jnp.mean(xc * xc, axis=-1, keepdims=True)
    return xc * lax.rsqrt(var + EPS) * g + b


def _node_body(h1_ref, ah_ref, ae_ref, wn1h_ref, wn1e_ref, bn1_ref, wn2_ref,
               bn2_ref, gh_ref, bth_ref, we1a_ref, we1b_ref,
               h2_ref, p_ref, q_ref):
    pre = (
        jnp.dot(ah_ref[...], wn1h_ref[...], preferred_element_type=F32)
        + jnp.dot(ae_ref[...], wn1e_ref[...], preferred_element_type=F32)
        + bn1_ref[...]
    )
    hid = jnp.maximum(pre, 0.0)
    h_new = jnp.maximum(
        jnp.dot(hid, wn2_ref[...], preferred_element_type=F32) + bn2_ref[...], 0.0
    )
    h2 = _layernorm(h1_ref[...] + h_new, gh_ref[...], bth_ref[...])
    h2_ref[...] = h2
    p_ref[...] = jnp.dot(h2, we1a_ref[...], preferred_element_type=F32)
    q_ref[...] = jnp.dot(h2, we1b_ref[...], preferred_element_type=F32)


def _node_finish(h1, ah, ae, wn1h, wn1e, bn1, wn2, bn2, gh, bth, we1a, we1b,
                 block):
    n, d = h1.shape
    assert n % block == 0
    row = lambda i: (i, 0)
    full = lambda i: (0, 0)
    return pl.pallas_call(
        _node_body,
        grid=(n // block,),
        in_specs=[
            pl.BlockSpec((block, d), row),
            pl.BlockSpec((block, d), row),
            pl.BlockSpec((block, d), row),
            pl.BlockSpec((d, d), full),
            pl.BlockSpec((d, d), full),
            pl.BlockSpec((1, d), full),
            pl.BlockSpec((d, d), full),
            pl.BlockSpec((1, d), full),
            pl.BlockSpec((1, d), full),
            pl.BlockSpec((1, d), full),
            pl.BlockSpec((d, d), full),
            pl.BlockSpec((d, d), full),
        ],
        out_specs=[
            pl.BlockSpec((block, d), row),
            pl.BlockSpec((block, d), row),
            pl.BlockSpec((block, d), row),
        ],
        out_shape=[
            jax.ShapeDtypeStruct((n, d), F32),
            jax.ShapeDtypeStruct((n, d), F32),
            jax.ShapeDtypeStruct((n, d), F32),
        ],
    )(h1, ah, ae, wn1h, wn1e, bn1, wn2, bn2, gh, bth, we1a, we1b)


def _edge_body(e1_ref, gp_ref, gq_ref, we1c_ref, be1_ref, we2_ref, be2_ref,
               ge_ref, bte_ref, e2_ref):
    e1 = e1_ref[...]
    t = (
        jnp.dot(e1, we1c_ref[...], preferred_element_type=F32)
        + gp_ref[...] + gq_ref[...] + be1_ref[...]
    )
    hid = jnp.maximum(t, 0.0)
    e_new = jnp.maximum(
        jnp.dot(hid, we2_ref[...], preferred_element_type=F32) + be2_ref[...], 0.0
    )
    e2_ref[...] = _layernorm(e1 + e_new, ge_ref[...], bte_ref[...])


def _edge_finish(e1, gp, gq, we1c, be1, we2, be2, ge, bte, block):
    e, d = e1.shape
    assert e % block == 0
    row = lambda i: (i, 0)
    full = lambda i: (0, 0)
    return pl.pallas_call(
        _edge_body,
        grid=(e // block,),
        in_specs=[
            pl.BlockSpec((block, d), row),
            pl.BlockSpec((block, d), row),
            pl.BlockSpec((block, d), row),
            pl.BlockSpec((d, d), full),
            pl.BlockSpec((1, d), full),
            pl.BlockSpec((d, d), full),
            pl.BlockSpec((1, d), full),
            pl.BlockSpec((1, d), full),
            pl.BlockSpec((1, d), full),
        ],
        out_specs=pl.BlockSpec((block, d), row),
        out_shape=jax.ShapeDtypeStruct((e, d), F32),
    )(e1, gp, gq, we1c, be1, we2, be2, ge, bte)


def _segment_sums_sc(h1, e1, src2, dst2):
    n, d = h1.shape
    e = src2.shape[1]
    assert e % WIN == 0
    nwin = e // WIN
    main_rows = (n // (8 * NUM_SUBCORES)) * 8
    tail_rows = n - main_rows * NUM_SUBCORES
    zch = 8
    assert main_rows % zch == 0 and tail_rows % zch == 0
    mesh = plsc.VectorSubcoreMesh(
        core_axis_name="core", subcore_axis_name="subcore"
    )

    @functools.partial(
        pl.kernel,
        out_type=(
            jax.ShapeDtypeStruct((n, d), F32),
            jax.ShapeDtypeStruct((n, d), F32),
        ),
        mesh=mesh,
        scratch_types=[
            pltpu.VMEM_SHARED((n, d), F32),
            pltpu.VMEM((WIN, d), F32),
            pltpu.VMEM((zch, d), F32),
        ],
    )
    def k(h1_hbm, e1_hbm, src_hbm, dst_hbm, ah_hbm, ae_hbm, acc, rows, zbuf):
        cid = lax.axis_index("core")
        sid = lax.axis_index("subcore")

        @pl.loop(0, zch)
        def _(i):
            for j in range(d // LANES):
                zbuf[i, pl.ds(j * LANES, LANES)] = jnp.zeros((LANES,), F32)

        base = pl.multiple_of(sid * main_rows, 8)

        @pl.loop(0, main_rows // zch)
        def _(kk):
            off = pl.multiple_of(base + kk * zch, 8)
            pltpu.sync_copy(zbuf, acc.at[pl.ds(off, zch)])

        if tail_rows:
            @pl.when(sid == 0)
            def _():
                @pl.loop(0, tail_rows // zch)
                def _(t):
                    off = pl.multiple_of(n - tail_rows + t * zch, 8)
                    pltpu.sync_copy(zbuf, acc.at[pl.ds(off, zch)])

        plsc.subcore_barrier()

        def body_h(sidx, didx):
            pltpu.sync_copy(h1_hbm.at[sidx.at[0]], rows)
            pltpu.sync_copy(rows, acc.at[didx.at[0]], add=True)

        def body_e(vals, didx):
            pltpu.sync_copy(vals, acc.at[didx.at[0]], add=True)

        @pl.when(cid == 0)
        def _():
            pltpu.emit_pipeline(
                body_h,
                grid=(nwin,),
                in_specs=[
                    pl.BlockSpec((1, WIN), lambda i: (0, i)),
                    pl.BlockSpec((1, WIN), lambda i: (0, i)),
                ],
                core_axis_name="subcore",
                dimension_semantics=(pltpu.PARALLEL,),
            )(src_hbm, dst_hbm)

        @pl.when(cid == 1)
        def _():
            pltpu.emit_pipeline(
                body_e,
                grid=(nwin,),
                in_specs=[
                    pl.BlockSpec((WIN, d), lambda i: (i, 0)),
                    pl.BlockSpec((1, WIN), lambda i: (0, i)),
                ],
                core_axis_name="subcore",
                dimension_semantics=(pltpu.PARALLEL,),
            )(e1_hbm, dst_hbm)

        plsc.subcore_barrier()

        def copy_out(out_hbm):
            pltpu.sync_copy(
                acc.at[pl.ds(base, main_rows)],
                out_hbm.at[pl.ds(base, main_rows)],
            )
            if tail_rows:
                @pl.when(sid == 0)
                def _():
                    pltpu.sync_copy(
                        acc.at[pl.ds(n - tail_rows, tail_rows)],
                        out_hbm.at[pl.ds(n - tail_rows, tail_rows)],
                    )

        @pl.when(cid == 0)
        def _():
            copy_out(ah_hbm)

        @pl.when(cid == 1)
        def _():
            copy_out(ae_hbm)

    return k(h1, e1, src2, dst2)


def _edge_gathers_sc(p, q, src2, dst2):
    n, d = p.shape
    e = src2.shape[1]
    assert e % WIN == 0
    nwin = e // WIN
    mesh = plsc.VectorSubcoreMesh(
        core_axis_name="core", subcore_axis_name="subcore"
    )

    @functools.partial(
        pl.kernel,
        out_type=(
            jax.ShapeDtypeStruct((e, d), F32),
            jax.ShapeDtypeStruct((e, d), F32),
        ),
        mesh=mesh,
        scratch_types=[],
    )
    def k(p_hbm, q_hbm, src_hbm, dst_hbm, gp_hbm, gq_hbm):
        cid = lax.axis_index("core")

        def gather_body_p(sidx, o_vmem):
            pltpu.sync_copy(p_hbm.at[sidx.at[0]], o_vmem)

        def gather_body_q(didx, o_vmem):
            pltpu.sync_copy(q_hbm.at[didx.at[0]], o_vmem)

        @pl.when(cid == 0)
        def _():
            pltpu.emit_pipeline(
                gather_body_p,
                grid=(nwin,),
                in_specs=[pl.BlockSpec((1, WIN), lambda i: (0, i))],
                out_specs=[pl.BlockSpec((WIN, d), lambda i: (i, 0))],
                core_axis_name="subcore",
                dimension_semantics=(pltpu.PARALLEL,),
            )(src_hbm, gp_hbm)

        @pl.when(cid == 1)
        def _():
            pltpu.emit_pipeline(
                gather_body_q,
                grid=(nwin,),
                in_specs=[pl.BlockSpec((1, WIN), lambda i: (0, i))],
                out_specs=[pl.BlockSpec((WIN, d), lambda i: (i, 0))],
                core_axis_name="subcore",
                dimension_semantics=(pltpu.PARALLEL,),
            )(dst_hbm, gq_hbm)

    return k(p, q, src2, dst2)


def kernel(h, e, edge_index, W_ph, b_ph, W_pe, b_pe, W_n1, b_n1, W_n2, b_n2,
           W_e1, b_e1, W_e2, b_e2, g_h, bt_h, g_e, bt_e):
    n, d = h.shape
    ne = e.shape[0]
    src2 = edge_index[0].reshape(1, ne)
    dst2 = edge_index[1].reshape(1, ne)

    b2 = lambda v: v.reshape(1, d)

    h1 = _prelin(h, W_ph, b2(b_ph), block=2000)
    e1 = _prelin(e, W_pe, b2(b_pe), block=4000)

    ah, ae = _segment_sums_sc(h1, e1, src2, dst2)

    h2, p, q = _node_finish(
        h1, ah, ae,
        W_n1[:d], W_n1[d:], b2(b_n1), W_n2, b2(b_n2), b2(g_h), b2(bt_h),
        W_e1[:d], W_e1[d:2 * d],
        block=2000,
    )

    gp, gq = _edge_gathers_sc(p, q, src2, dst2)

    e2 = _edge_finish(
        e1, gp, gq, W_e1[2 * d:], b2(b_e1), W_e2, b2(b_e2), b2(g_e), b2(bt_e),
        block=4000,
    )
    return (h2, e2)


# speedup vs baseline: 5.5898x; 1.1755x over previous
import functools

import jax
import jax.numpy as jnp
from jax import lax
from jax.experimental import pallas as pl
from jax.experimental.pallas import tpu as pltpu
from jax.experimental.pallas import tpu_sc as plsc

EPS = 1e-5
F32 = jnp.float32

NUM_CORES = 2
NUM_SUBCORES = 16
LANES = 16
WIN = 128


def _prelin_body(x_ref, w_ref, b_ref, o_ref):
    o_ref[...] = (
        jnp.dot(x_ref[...], w_ref[...], preferred_element_type=F32) + b_ref[...]
    )


def _prelin(x, w, b, block):
    n, d = x.shape
    assert n % block == 0
    return pl.pallas_call(
        _prelin_body,
        grid=(n // block,),
        in_specs=[
            pl.BlockSpec((block, d), lambda i: (i, 0)),
            pl.BlockSpec((d, d), lambda i: (0, 0)),
            pl.BlockSpec((1, d), lambda i: (0, 0)),
        ],
        out_specs=pl.BlockSpec((block, d), lambda i: (i, 0)),
        out_shape=jax.ShapeDtypeStruct((n, d), F32),
    )(x, w, b)


def _layernorm(x, g, b):
    mu = jnp.mean(x, axis=-1, keepdims=True)
    xc = x - mu
    var = jnp.mean(xc * xc, axis=-1, keepdims=True)
    return xc * lax.rsqrt(var + EPS) * g + b


def _node_body(h1_ref, ah_ref, ae_ref, wn1h_ref, wn1e_ref, bn1_ref, wn2_ref,
               bn2_ref, gh_ref, bth_ref, we1a_ref, we1b_ref,
               h2_ref, p_ref, q_ref):
    pre = (
        jnp.dot(ah_ref[...], wn1h_ref[...], preferred_element_type=F32)
        + jnp.dot(ae_ref[...], wn1e_ref[...], preferred_element_type=F32)
        + bn1_ref[...]
    )
    hid = jnp.maximum(pre, 0.0)
    h_new = jnp.maximum(
        jnp.dot(hid, wn2_ref[...], preferred_element_type=F32) + bn2_ref[...], 0.0
    )
    h2 = _layernorm(h1_ref[...] + h_new, gh_ref[...], bth_ref[...])
    h2_ref[...] = h2
    p_ref[...] = jnp.dot(h2, we1a_ref[...], preferred_element_type=F32)
    q_ref[...] = jnp.dot(h2, we1b_ref[...], preferred_element_type=F32)


def _node_finish(h1, ah, ae, wn1h, wn1e, bn1, wn2, bn2, gh, bth, we1a, we1b,
                 block):
    n, d = h1.shape
    assert n % block == 0
    row = lambda i: (i, 0)
    full = lambda i: (0, 0)
    return pl.pallas_call(
        _node_body,
        grid=(n // block,),
        in_specs=[
            pl.BlockSpec((block, d), row),
            pl.BlockSpec((block, d), row),
            pl.BlockSpec((block, d), row),
            pl.BlockSpec((d, d), full),
            pl.BlockSpec((d, d), full),
            pl.BlockSpec((1, d), full),
            pl.BlockSpec((d, d), full),
            pl.BlockSpec((1, d), full),
            pl.BlockSpec((1, d), full),
            pl.BlockSpec((1, d), full),
            pl.BlockSpec((d, d), full),
            pl.BlockSpec((d, d), full),
        ],
        out_specs=[
            pl.BlockSpec((block, d), row),
            pl.BlockSpec((block, d), row),
            pl.BlockSpec((block, d), row),
        ],
        out_shape=[
            jax.ShapeDtypeStruct((n, d), F32),
            jax.ShapeDtypeStruct((n, d), F32),
            jax.ShapeDtypeStruct((n, d), F32),
        ],
    )(h1, ah, ae, wn1h, wn1e, bn1, wn2, bn2, gh, bth, we1a, we1b)


def _edge_body(e1_ref, gp_ref, gq_ref, we1c_ref, be1_ref, we2_ref, be2_ref,
               ge_ref, bte_ref, e2_ref):
    e1 = e1_ref[...]
    t = (
        jnp.dot(e1, we1c_ref[...], preferred_element_type=F32)
        + gp_ref[...] + gq_ref[...] + be1_ref[...]
    )
    hid = jnp.maximum(t, 0.0)
    e_new = jnp.maximum(
        jnp.dot(hid, we2_ref[...], preferred_element_type=F32) + be2_ref[...], 0.0
    )
    e2_ref[...] = _layernorm(e1 + e_new, ge_ref[...], bte_ref[...])


def _edge_finish(e1, gp, gq, we1c, be1, we2, be2, ge, bte, block):
    e, d = e1.shape
    assert e % block == 0
    row = lambda i: (i, 0)
    full = lambda i: (0, 0)
    return pl.pallas_call(
        _edge_body,
        grid=(e // block,),
        in_specs=[
            pl.BlockSpec((block, d), row),
            pl.BlockSpec((block, d), row),
            pl.BlockSpec((block, d), row),
            pl.BlockSpec((d, d), full),
            pl.BlockSpec((1, d), full),
            pl.BlockSpec((d, d), full),
            pl.BlockSpec((1, d), full),
            pl.BlockSpec((1, d), full),
            pl.BlockSpec((1, d), full),
        ],
        out_specs=pl.BlockSpec((block, d), row),
        out_shape=jax.ShapeDtypeStruct((e, d), F32),
    )(e1, gp, gq, we1c, be1, we2, be2, ge, bte)


def _segment_sums_sc(h1, e1, src2, dst2):
    n, d = h1.shape
    e = src2.shape[1]
    assert e % WIN == 0
    nwin = e // WIN
    main_rows = (n // (8 * NUM_SUBCORES)) * 8
    tail_rows = n - main_rows * NUM_SUBCORES
    zch = 8
    assert main_rows % zch == 0 and tail_rows % zch == 0
    mesh = plsc.VectorSubcoreMesh(
        core_axis_name="core", subcore_axis_name="subcore"
    )

    @functools.partial(
        pl.kernel,
        out_type=(
            jax.ShapeDtypeStruct((n, d), F32),
            jax.ShapeDtypeStruct((n, d), F32),
        ),
        mesh=mesh,
        scratch_types=[
            pltpu.VMEM_SHARED((n, d), F32),
            pltpu.VMEM((WIN, d), F32),
            pltpu.VMEM((zch, d), F32),
        ],
    )
    def k(h1_hbm, e1_hbm, src_hbm, dst_hbm, ah_hbm, ae_hbm, acc, rows, zbuf):
        cid = lax.axis_index("core")
        sid = lax.axis_index("subcore")

        @pl.loop(0, zch)
        def _(i):
            for j in range(d // LANES):
                zbuf[i, pl.ds(j * LANES, LANES)] = jnp.zeros((LANES,), F32)

        base = pl.multiple_of(sid * main_rows, 8)

        @pl.loop(0, main_rows // zch)
        def _(kk):
            off = pl.multiple_of(base + kk * zch, 8)
            pltpu.sync_copy(zbuf, acc.at[pl.ds(off, zch)])

        if tail_rows:
            @pl.when(sid == 0)
            def _():
                @pl.loop(0, tail_rows // zch)
                def _(t):
                    off = pl.multiple_of(n - tail_rows + t * zch, 8)
                    pltpu.sync_copy(zbuf, acc.at[pl.ds(off, zch)])

        plsc.subcore_barrier()

        def body_h(sidx, didx):
            pltpu.sync_copy(h1_hbm.at[sidx.at[0]], rows)
            pltpu.sync_copy(rows, acc.at[didx.at[0]], add=True)

        def body_e(vals, didx):
            pltpu.sync_copy(vals, acc.at[didx.at[0]], add=True)

        @pl.when(cid == 0)
        def _():
            pltpu.emit_pipeline(
                body_h,
                grid=(nwin,),
                in_specs=[
                    pl.BlockSpec((1, WIN), lambda i: (0, i)),
                    pl.BlockSpec((1, WIN), lambda i: (0, i)),
                ],
                core_axis_name="subcore",
                dimension_semantics=(pltpu.PARALLEL,),
            )(src_hbm, dst_hbm)

        @pl.when(cid == 1)
        def _():
            pltpu.emit_pipeline(
                body_e,
                grid=(nwin,),
                in_specs=[
                    pl.BlockSpec((WIN, d), lambda i: (i, 0)),
                    pl.BlockSpec((1, WIN), lambda i: (0, i)),
                ],
                core_axis_name="subcore",
                dimension_semantics=(pltpu.PARALLEL,),
            )(e1_hbm, dst_hbm)

        plsc.subcore_barrier()

        def copy_out(out_hbm):
            pltpu.sync_copy(
                acc.at[pl.ds(base, main_rows)],
                out_hbm.at[pl.ds(base, main_rows)],
            )
            if tail_rows:
                @pl.when(sid == 0)
                def _():
                    pltpu.sync_copy(
                        acc.at[pl.ds(n - tail_rows, tail_rows)],
                        out_hbm.at[pl.ds(n - tail_rows, tail_rows)],
                    )

        @pl.when(cid == 0)
        def _():
            copy_out(ah_hbm)

        @pl.when(cid == 1)
        def _():
            copy_out(ae_hbm)

    return k(h1, e1, src2, dst2)


def _edge_gathers_sc(p, q, src2, dst2):
    n, d = p.shape
    e = src2.shape[1]
    assert e % WIN == 0
    nwin = e // WIN
    main_rows = (n // (8 * NUM_SUBCORES)) * 8
    tail_rows = n - main_rows * NUM_SUBCORES
    mesh = plsc.VectorSubcoreMesh(
        core_axis_name="core", subcore_axis_name="subcore"
    )

    @functools.partial(
        pl.kernel,
        out_type=(
            jax.ShapeDtypeStruct((e, d), p.dtype),
            jax.ShapeDtypeStruct((e, d), p.dtype),
        ),
        mesh=mesh,
        scratch_types=[
            pltpu.VMEM_SHARED((n, d), F32),
        ],
    )
    def k(p_hbm, q_hbm, src_hbm, dst_hbm, gp_hbm, gq_hbm, tab):
        cid = lax.axis_index("core")
        sid = lax.axis_index("subcore")
        base = pl.multiple_of(sid * main_rows, 8)

        def stage(src_hbm_tab):
            pltpu.sync_copy(
                src_hbm_tab.at[pl.ds(base, main_rows)],
                tab.at[pl.ds(base, main_rows)],
            )
            if tail_rows:
                @pl.when(sid == 0)
                def _():
                    pltpu.sync_copy(
                        src_hbm_tab.at[pl.ds(n - tail_rows, tail_rows)],
                        tab.at[pl.ds(n - tail_rows, tail_rows)],
                    )

        @pl.when(cid == 0)
        def _():
            stage(p_hbm)

        @pl.when(cid == 1)
        def _():
            stage(q_hbm)

        plsc.subcore_barrier()

        def gather_body_p(sidx, o_vmem):
            pltpu.sync_copy(tab.at[sidx.at[0]], o_vmem)

        def gather_body_q(didx, o_vmem):
            pltpu.sync_copy(tab.at[didx.at[0]], o_vmem)

        @pl.when(cid == 0)
        def _():
            pltpu.emit_pipeline(
                gather_body_p,
                grid=(nwin,),
                in_specs=[pl.BlockSpec((1, WIN), lambda i: (0, i))],
                out_specs=[pl.BlockSpec((WIN, d), lambda i: (i, 0))],
                core_axis_name="subcore",
                dimension_semantics=(pltpu.PARALLEL,),
            )(src_hbm, gp_hbm)

        @pl.when(cid == 1)
        def _():
            pltpu.emit_pipeline(
                gather_body_q,
                grid=(nwin,),
                in_specs=[pl.BlockSpec((1, WIN), lambda i: (0, i))],
                out_specs=[pl.BlockSpec((WIN, d), lambda i: (i, 0))],
                core_axis_name="subcore",
                dimension_semantics=(pltpu.PARALLEL,),
            )(dst_hbm, gq_hbm)

    return k(p, q, src2, dst2)


def kernel(h, e, edge_index, W_ph, b_ph, W_pe, b_pe, W_n1, b_n1, W_n2, b_n2,
           W_e1, b_e1, W_e2, b_e2, g_h, bt_h, g_e, bt_e):
    n, d = h.shape
    ne = e.shape[0]
    src2 = edge_index[0].reshape(1, ne)
    dst2 = edge_index[1].reshape(1, ne)

    b2 = lambda v: v.reshape(1, d)

    h1 = _prelin(h, W_ph, b2(b_ph), block=2000)
    e1 = _prelin(e, W_pe, b2(b_pe), block=4000)

    ah, ae = _segment_sums_sc(h1, e1, src2, dst2)

    h2, p, q = _node_finish(
        h1, ah, ae,
        W_n1[:d], W_n1[d:], b2(b_n1), W_n2, b2(b_n2), b2(g_h), b2(bt_h),
        W_e1[:d], W_e1[d:2 * d],
        block=2000,
    )

    gp, gq = _edge_gathers_sc(p, q, src2, dst2)

    e2 = _edge_finish(
        e1, gp, gq, W_e1[2 * d:], b2(b_e1), W_e2, b2(b_e2), b2(g_e), b2(bt_e),
        block=4000,
    )
    return (h2, e2)
